# Initial kernel scaffold; baseline (speedup 1.0000x reference)
import jax
import jax.numpy as jnp
from jax.experimental import pallas as pl


def kernel(hidden_states, W1, b1, W2, b2, Wc, bc, Wg1, bg1, Wg2, bg2, expert_scales, expert_biases):
    raise NotImplementedError("write your pallas kernel here")


# speedup vs baseline: 2.8201x; 2.8201x over previous
import jax
import jax.numpy as jnp
from jax.experimental import pallas as pl

_BT = 512


def _router_body(x_ref, w1_ref, b1_ref, w2_ref, b2_ref, es_ref, eb_ref, o_ref):
    x = x_ref[...]
    h = jnp.dot(x, w1_ref[...], preferred_element_type=jnp.float32)
    h = jnp.maximum(h + b1_ref[...], 0.0)
    s = jnp.dot(h, w2_ref[...], preferred_element_type=jnp.float32) + b2_ref[...]
    num_e = s.shape[1]
    m = jnp.max(s, axis=1, keepdims=True)
    ii = jax.lax.broadcasted_iota(jnp.int32, s.shape, 1)
    idx = jnp.min(jnp.where(s == m, ii, num_e), axis=1, keepdims=True)
    onehot = (ii == idx).astype(jnp.float32)
    scale = jnp.dot(onehot, es_ref[...], preferred_element_type=jnp.float32)
    bias = jnp.dot(onehot, eb_ref[...], preferred_element_type=jnp.float32)
    o_ref[...] = x * scale + bias


def kernel(hidden_states, W1, b1, W2, b2, Wc, bc, Wg1, bg1, Wg2, bg2,
           expert_scales, expert_biases):
    B, S, H = hidden_states.shape
    RH = W1.shape[1]
    E = expert_scales.shape[0]
    N = B * S
    flat = hidden_states.reshape(N, H)
    out = pl.pallas_call(
        _router_body,
        grid=(N // _BT,),
        in_specs=[
            pl.BlockSpec((_BT, H), lambda i: (i, 0)),
            pl.BlockSpec((H, RH), lambda i: (0, 0)),
            pl.BlockSpec((1, RH), lambda i: (0, 0)),
            pl.BlockSpec((RH, E), lambda i: (0, 0)),
            pl.BlockSpec((1, E), lambda i: (0, 0)),
            pl.BlockSpec((E, H), lambda i: (0, 0)),
            pl.BlockSpec((E, H), lambda i: (0, 0)),
        ],
        out_specs=pl.BlockSpec((_BT, H), lambda i: (i, 0)),
        out_shape=jax.ShapeDtypeStruct((N, H), jnp.float32),
    )(flat, W1, b1.reshape(1, RH), W2, b2.reshape(1, E),
      expert_scales, expert_biases)
    return out.reshape(B, S, H)


# speedup vs baseline: 2.9415x; 1.0431x over previous
import jax
import jax.numpy as jnp
from jax.experimental import pallas as pl

_BT = 1024


def _router_body(x_ref, w1_ref, b1_ref, w2_ref, b2_ref, esb_ref, o_ref):
    x = x_ref[...]
    h = jnp.dot(x, w1_ref[...], preferred_element_type=jnp.float32)
    h = jnp.maximum(h + b1_ref[...], 0.0)
    s = jnp.dot(h, w2_ref[...], preferred_element_type=jnp.float32) + b2_ref[...]
    num_e = s.shape[1]
    m = jnp.max(s, axis=1, keepdims=True)
    ii = jax.lax.broadcasted_iota(jnp.int32, s.shape, 1)
    idx = jnp.min(jnp.where(s == m, ii, num_e), axis=1, keepdims=True)
    onehot = (ii == idx).astype(jnp.float32)
    sb = jnp.dot(onehot, esb_ref[...], preferred_element_type=jnp.float32)
    h_dim = x.shape[1]
    o_ref[...] = x * sb[:, :h_dim] + sb[:, h_dim:]


def kernel(hidden_states, W1, b1, W2, b2, Wc, bc, Wg1, bg1, Wg2, bg2,
           expert_scales, expert_biases):
    B, S, H = hidden_states.shape
    RH = W1.shape[1]
    E = expert_scales.shape[0]
    N = B * S
    flat = hidden_states.reshape(N, H)
    esb = jnp.concatenate([expert_scales, expert_biases], axis=1)
    out = pl.pallas_call(
        _router_body,
        grid=(N // _BT,),
        in_specs=[
            pl.BlockSpec((_BT, H), lambda i: (i, 0)),
            pl.BlockSpec((H, RH), lambda i: (0, 0)),
            pl.BlockSpec((1, RH), lambda i: (0, 0)),
            pl.BlockSpec((RH, E), lambda i: (0, 0)),
            pl.BlockSpec((1, E), lambda i: (0, 0)),
            pl.BlockSpec((E, 2 * H), lambda i: (0, 0)),
        ],
        out_specs=pl.BlockSpec((_BT, H), lambda i: (i, 0)),
        out_shape=jax.ShapeDtypeStruct((N, H), jnp.float32),
    )(flat, W1, b1.reshape(1, RH), W2, b2.reshape(1, E), esb)
    return out.reshape(B, S, H)
